# baseline (speedup 1.0000x reference)
import functools

import jax
import jax.numpy as jnp
from jax import lax
from jax.experimental import pallas as pl
from jax.experimental.pallas import tpu as pltpu
from jax.experimental.pallas import tpu_sc as plsc

N = 10000
F = 128
E = 320000
NC = 2
NS = 16
NW = NC * NS
C = 120
NCH = 84
NIB = 6
NRB = 3
EPT = NCH * C
E_PAD = NW * EPT
NP = 10240
RPT = NP // NS
NACC = 10112
RPTA = NACC // NS


def _deg_body(dstf, degp, dacc, ones_v, zeros_v, *sc):
    idxd = sc[0:NIB]
    semi = sc[NIB:2 * NIB]
    semd = sc[2 * NIB:3 * NIB]
    c = lax.axis_index("c")
    s = lax.axis_index("s")
    w = s * NC + c
    for i in range(C // 16):
        ones_v[pl.ds(i * 16, 16)] = jnp.ones((16,), jnp.float32)
    if C % 16:
        ones_v[pl.ds(C - 16, 16)] = jnp.ones((16,), jnp.float32)
    for i in range(RPT // 16):
        zeros_v[pl.ds(i * 16, 16)] = jnp.zeros((16,), jnp.float32)
    pltpu.sync_copy(zeros_v, dacc.at[pl.ds(s * RPT, RPT)])
    plsc.subcore_barrier()

    for b in range(3):
        pltpu.async_copy(dstf.at[pl.ds((w * NCH + b) * C, C)],
                         idxd[b].at[0], semi[b])

    @pl.loop(0, NCH // 6)
    def _(jg):
        for u in range(6):
            j = jg * 6 + u
            pltpu.make_async_copy(dstf.at[pl.ds(0, C)], idxd[u].at[0],
                                  semi[u]).wait()

            @pl.when(j >= 3)
            def _():
                pltpu.make_async_copy(ones_v, dacc.at[idxd[(u + 3) % 6].at[0]],
                                      semd[(u + 3) % 6]).wait()

            pltpu.async_copy(ones_v, dacc.at[idxd[u].at[0]], semd[u],
                             add=True)

            @pl.when(j + 3 < NCH)
            def _():
                pltpu.async_copy(dstf.at[pl.ds((w * NCH + j + 3) * C, C)],
                                 idxd[(u + 3) % NIB].at[0],
                                 semi[(u + 3) % NIB])

    for u in range(3):
        pltpu.make_async_copy(ones_v, dacc.at[idxd[u].at[0]],
                              semd[(NCH - 3 + u) % 6]).wait()

    plsc.subcore_barrier()
    pltpu.sync_copy(dacc.at[pl.ds(s * RPT, RPT)],
                    degp.at[c, pl.ds(s * RPT, RPT)])


@functools.lru_cache(maxsize=None)
def _deg_kernel():
    mesh = plsc.VectorSubcoreMesh(core_axis_name="c", subcore_axis_name="s",
                                  num_cores=NC, num_subcores=NS)
    return pl.kernel(
        _deg_body,
        out_type=jax.ShapeDtypeStruct((NC, NP), jnp.float32),
        mesh=mesh,
        scratch_types=(
            [pltpu.VMEM_SHARED((NP,), jnp.float32),
             pltpu.VMEM((C,), jnp.float32),
             pltpu.VMEM((RPT,), jnp.float32)]
            + [pltpu.VMEM((1, C), jnp.int32) for _ in range(NIB)]
            + [pltpu.SemaphoreType.DMA for _ in range(2 * NIB)]
        ),
    )


def _agg_body(xs, srcf, dstf, part, acc, *sc):
    idxs = sc[0:NIB]
    idxd = sc[NIB:2 * NIB]
    rows = sc[2 * NIB:2 * NIB + NRB]
    semi = sc[2 * NIB + NRB:3 * NIB + NRB]
    semg = sc[3 * NIB + NRB:3 * NIB + 2 * NRB]
    sems = sc[3 * NIB + 2 * NRB:3 * NIB + 3 * NRB]
    c = lax.axis_index("c")
    s = lax.axis_index("s")
    w = s * NC + c

    @pl.loop(0, C)
    def _(r):
        for i in range(F // 16):
            rows[0][r, pl.ds(i * 16, 16)] = jnp.zeros((16,), jnp.float32)

    for r in range(RPTA // C):
        pltpu.sync_copy(rows[0], acc.at[pl.ds(s * RPTA + r * C, C)])
    pltpu.sync_copy(rows[0].at[pl.ds(0, RPTA % C)],
                    acc.at[pl.ds(s * RPTA + (RPTA // C) * C, RPTA % C)])
    plsc.subcore_barrier()

    for b in range(3):
        pltpu.async_copy(srcf.at[pl.ds((w * NCH + b) * C, C)],
                         idxs[b].at[0], semi[b])
        pltpu.async_copy(dstf.at[pl.ds((w * NCH + b) * C, C)],
                         idxd[b].at[0], semi[b])

    @pl.loop(0, NCH // 6)
    def _(jg):
        j0 = jg * 6
        for u in range(6):
            j = j0 + u
            b3 = u % 3
            pltpu.make_async_copy(srcf.at[pl.ds(0, C)], idxs[u].at[0],
                                  semi[u]).wait()
            pltpu.make_async_copy(dstf.at[pl.ds(0, C)], idxd[u].at[0],
                                  semi[u]).wait()

            @pl.when(j >= 3)
            def _():
                pltpu.make_async_copy(rows[b3], acc.at[idxd[u].at[0]],
                                      sems[b3]).wait()

            pltpu.async_copy(xs.at[idxs[u].at[0]], rows[b3], semg[b3])

            @pl.when(j + 3 < NCH)
            def _():
                pltpu.async_copy(srcf.at[pl.ds((w * NCH + j + 3) * C, C)],
                                 idxs[(u + 3) % NIB].at[0],
                                 semi[(u + 3) % NIB])
                pltpu.async_copy(dstf.at[pl.ds((w * NCH + j + 3) * C, C)],
                                 idxd[(u + 3) % NIB].at[0],
                                 semi[(u + 3) % NIB])

            up = (u + 5) % 6
            bp = (u + 2) % 3

            @pl.when(j >= 1)
            def _():
                pltpu.make_async_copy(xs.at[idxs[up].at[0]], rows[bp],
                                      semg[bp]).wait()
                pltpu.async_copy(rows[bp], acc.at[idxd[up].at[0]], sems[bp],
                                 add=True)

    pltpu.make_async_copy(xs.at[idxs[(NCH - 1) % 6].at[0]],
                          rows[(NCH - 1) % 3], semg[(NCH - 1) % 3]).wait()
    pltpu.async_copy(rows[(NCH - 1) % 3],
                     acc.at[idxd[(NCH - 1) % 6].at[0]],
                     sems[(NCH - 1) % 3], add=True)
    for b in range(3):
        pltpu.make_async_copy(rows[b], acc.at[idxd[b].at[0]], sems[b]).wait()

    plsc.subcore_barrier()
    pltpu.sync_copy(acc.at[pl.ds(s * RPTA, RPTA)],
                    part.at[c, pl.ds(s * RPTA, RPTA)])


@functools.lru_cache(maxsize=None)
def _agg_kernel():
    mesh = plsc.VectorSubcoreMesh(core_axis_name="c", subcore_axis_name="s",
                                  num_cores=NC, num_subcores=NS)
    return pl.kernel(
        _agg_body,
        out_type=jax.ShapeDtypeStruct((NC, NP, F), jnp.float32),
        mesh=mesh,
        scratch_types=(
            [pltpu.VMEM_SHARED((NACC, F), jnp.float32)]
            + [pltpu.VMEM((1, C), jnp.int32) for _ in range(2 * NIB)]
            + [pltpu.VMEM((C, F), jnp.float32) for _ in range(NRB)]
            + [pltpu.SemaphoreType.DMA for _ in range(NIB + 2 * NRB)]
        ),
    )


_BM = 1024
_GRID = -(-N // _BM)
_DB = _BM // 128


def _dcol(dv):
    g = lax.broadcasted_iota(jnp.int32, (_BM, _DB), 0) // 128
    h = lax.broadcasted_iota(jnp.int32, (_BM, _DB), 1)
    rep = jnp.dot((g == h).astype(jnp.float32), dv,
                  preferred_element_type=jnp.float32)
    ri = lax.broadcasted_iota(jnp.int32, (_BM, 128), 0)
    li = lax.broadcasted_iota(jnp.int32, (_BM, 128), 1)
    pick = li == (ri % 128)
    return jnp.sum(jnp.where(pick, rep, 0.0), axis=1, keepdims=True)


def _scale_body(degp, x, dinv, xs):
    d = degp[0] + degp[1] + 1.0
    dv = lax.rsqrt(d)
    dinv[...] = dv
    xs[...] = x[...] * _dcol(dv)


def _scale_call(degp, x):
    return pl.pallas_call(
        _scale_body,
        grid=(_GRID,),
        in_specs=[
            pl.BlockSpec((NC, _DB, 128), lambda i: (0, i, 0)),
            pl.BlockSpec((_BM, F), lambda i: (i, 0)),
        ],
        out_specs=[
            pl.BlockSpec((_DB, 128), lambda i: (i, 0)),
            pl.BlockSpec((_BM, F), lambda i: (i, 0)),
        ],
        out_shape=[
            jax.ShapeDtypeStruct((NP // 128, 128), jnp.float32),
            jax.ShapeDtypeStruct((N, F), jnp.float32),
        ],
    )(degp, x)


def _mid_body(part, xs, dinv, W1, b1, W2, ts):
    dv = _dcol(dinv[...])
    agg = (part[0] + part[1] + xs[...]) * dv
    h1 = jnp.maximum(
        jnp.dot(agg, W1[...], preferred_element_type=jnp.float32) + b1[...],
        0.0)
    t = jnp.dot(h1, W2[...], preferred_element_type=jnp.float32)
    ts[...] = t * dv


def _mid_call(part, xs, dinv, W1, b1, W2):
    return pl.pallas_call(
        _mid_body,
        grid=(_GRID,),
        in_specs=[
            pl.BlockSpec((NC, _BM, F), lambda i: (0, i, 0)),
            pl.BlockSpec((_BM, F), lambda i: (i, 0)),
            pl.BlockSpec((_DB, 128), lambda i: (i, 0)),
            pl.BlockSpec((F, 2 * F), lambda i: (0, 0)),
            pl.BlockSpec((1, 2 * F), lambda i: (0, 0)),
            pl.BlockSpec((2 * F, F), lambda i: (0, 0)),
        ],
        out_specs=pl.BlockSpec((_BM, F), lambda i: (i, 0)),
        out_shape=jax.ShapeDtypeStruct((N, F), jnp.float32),
    )(part, xs, dinv, W1, b1, W2)


def _fin_body(part, ts, dinv, b2, Wl, bl, out):
    dv = _dcol(dinv[...])
    h2 = jnp.maximum((part[0] + part[1] + ts[...]) * dv + b2[...], 0.0)
    lg = jnp.dot(h2, Wl[...], preferred_element_type=jnp.float32) + bl[...]
    m = jnp.max(lg, axis=-1, keepdims=True)
    lse = m + jnp.log(jnp.sum(jnp.exp(lg - m), axis=-1, keepdims=True))
    out[...] = lg - lse


def _fin_call(part, ts, dinv, b2, Wl, bl):
    return pl.pallas_call(
        _fin_body,
        grid=(_GRID,),
        in_specs=[
            pl.BlockSpec((NC, _BM, F), lambda i: (0, i, 0)),
            pl.BlockSpec((_BM, F), lambda i: (i, 0)),
            pl.BlockSpec((_DB, 128), lambda i: (i, 0)),
            pl.BlockSpec((1, F), lambda i: (0, 0)),
            pl.BlockSpec((F, 2), lambda i: (0, 0)),
            pl.BlockSpec((1, 2), lambda i: (0, 0)),
        ],
        out_specs=pl.BlockSpec((_BM, 2), lambda i: (i, 0)),
        out_shape=jax.ShapeDtypeStruct((N, 2), jnp.float32),
    )(part, ts, dinv, b2, Wl, bl)


def kernel(x, edge_index, W1, b1, W2, b2, Wl, bl):
    src = edge_index[0]
    dst = edge_index[1]
    pad = E_PAD - E
    ar = jnp.arange(pad, dtype=jnp.int32)
    srcf = jnp.concatenate([src, ar % 128])
    dstf = jnp.concatenate([dst, N + (ar % 32)])

    degp = _deg_kernel()(dstf)
    dinv, xs = _scale_call(degp.reshape(NC, NP // 128, 128), x)

    part1 = _agg_kernel()(xs, srcf, dstf)
    ts = _mid_call(part1, xs, dinv, W1, b1.reshape(1, 2 * F), W2)

    part2 = _agg_kernel()(ts, srcf, dstf)
    return _fin_call(part2, ts, dinv, b2.reshape(1, F), Wl, bl.reshape(1, 2))


# speedup vs baseline: 1.0258x; 1.0258x over previous
import functools

import jax
import jax.numpy as jnp
from jax import lax
from jax.experimental import pallas as pl
from jax.experimental.pallas import tpu as pltpu
from jax.experimental.pallas import tpu_sc as plsc

N = 10000
F = 128
E = 320000
NC = 2
NS = 16
NW = NC * NS
C = 120
NCH = 84
NIB = 6
NRB = 3
EPT = NCH * C
E_PAD = NW * EPT
NP = 10240
RPT = NP // NS
NACC = 10112
RPTA = NACC // NS


def _deg_body(dstf, degp, dacc, ones_v, zeros_v, *sc):
    idxd = sc[0:12]
    semi = sc[12:24]
    semd = sc[24:36]
    c = lax.axis_index("c")
    s = lax.axis_index("s")
    w = s * NC + c
    for i in range(C // 16):
        ones_v[pl.ds(i * 16, 16)] = jnp.ones((16,), jnp.float32)
    if C % 16:
        ones_v[pl.ds(C - 16, 16)] = jnp.ones((16,), jnp.float32)
    for i in range(RPT // 16):
        zeros_v[pl.ds(i * 16, 16)] = jnp.zeros((16,), jnp.float32)
    pltpu.sync_copy(zeros_v, dacc.at[pl.ds(s * RPT, RPT)])
    plsc.subcore_barrier()

    for b in range(6):
        pltpu.async_copy(dstf.at[pl.ds((w * NCH + b) * C, C)],
                         idxd[b].at[0], semi[b])

    @pl.loop(0, NCH // 12)
    def _(jg):
        for u in range(12):
            j = jg * 12 + u
            pltpu.make_async_copy(dstf.at[pl.ds(0, C)], idxd[u].at[0],
                                  semi[u]).wait()

            @pl.when(j >= 6)
            def _():
                pltpu.make_async_copy(ones_v,
                                      dacc.at[idxd[(u + 6) % 12].at[0]],
                                      semd[(u + 6) % 12]).wait()

            pltpu.async_copy(ones_v, dacc.at[idxd[u].at[0]], semd[u],
                             add=True)

            @pl.when(j + 6 < NCH)
            def _():
                pltpu.async_copy(dstf.at[pl.ds((w * NCH + j + 6) * C, C)],
                                 idxd[(u + 6) % 12].at[0],
                                 semi[(u + 6) % 12])

    for u in range(6):
        pltpu.make_async_copy(ones_v, dacc.at[idxd[u].at[0]],
                              semd[(NCH - 6 + u) % 12]).wait()

    plsc.subcore_barrier()
    pltpu.sync_copy(dacc.at[pl.ds(s * RPT, RPT)],
                    degp.at[c, pl.ds(s * RPT, RPT)])


@functools.lru_cache(maxsize=None)
def _deg_kernel():
    mesh = plsc.VectorSubcoreMesh(core_axis_name="c", subcore_axis_name="s",
                                  num_cores=NC, num_subcores=NS)
    return pl.kernel(
        _deg_body,
        out_type=jax.ShapeDtypeStruct((NC, NP), jnp.float32),
        mesh=mesh,
        scratch_types=(
            [pltpu.VMEM_SHARED((NP,), jnp.float32),
             pltpu.VMEM((C,), jnp.float32),
             pltpu.VMEM((RPT,), jnp.float32)]
            + [pltpu.VMEM((1, C), jnp.int32) for _ in range(12)]
            + [pltpu.SemaphoreType.DMA for _ in range(24)]
        ),
    )


def _agg_body(xs, srcf, dstf, part, acc, *sc):
    idxs = sc[0:NIB]
    idxd = sc[NIB:2 * NIB]
    rows = sc[2 * NIB:2 * NIB + NRB]
    semi = sc[2 * NIB + NRB:3 * NIB + NRB]
    semg = sc[3 * NIB + NRB:3 * NIB + 2 * NRB]
    sems = sc[3 * NIB + 2 * NRB:3 * NIB + 3 * NRB]
    c = lax.axis_index("c")
    s = lax.axis_index("s")
    w = s * NC + c

    @pl.loop(0, C)
    def _(r):
        for i in range(F // 16):
            rows[0][r, pl.ds(i * 16, 16)] = jnp.zeros((16,), jnp.float32)

    for r in range(RPTA // C):
        pltpu.sync_copy(rows[0], acc.at[pl.ds(s * RPTA + r * C, C)])
    pltpu.sync_copy(rows[0].at[pl.ds(0, RPTA % C)],
                    acc.at[pl.ds(s * RPTA + (RPTA // C) * C, RPTA % C)])
    plsc.subcore_barrier()

    for b in range(3):
        pltpu.async_copy(srcf.at[pl.ds((w * NCH + b) * C, C)],
                         idxs[b].at[0], semi[b])
        pltpu.async_copy(dstf.at[pl.ds((w * NCH + b) * C, C)],
                         idxd[b].at[0], semi[b])

    @pl.loop(0, NCH // 6)
    def _(jg):
        j0 = jg * 6
        for u in range(6):
            j = j0 + u
            b3 = u % 3
            pltpu.make_async_copy(srcf.at[pl.ds(0, C)], idxs[u].at[0],
                                  semi[u]).wait()
            pltpu.make_async_copy(dstf.at[pl.ds(0, C)], idxd[u].at[0],
                                  semi[u]).wait()

            @pl.when(j >= 3)
            def _():
                pltpu.make_async_copy(rows[b3], acc.at[idxd[u].at[0]],
                                      sems[b3]).wait()

            pltpu.async_copy(xs.at[idxs[u].at[0]], rows[b3], semg[b3])

            @pl.when(j + 3 < NCH)
            def _():
                pltpu.async_copy(srcf.at[pl.ds((w * NCH + j + 3) * C, C)],
                                 idxs[(u + 3) % NIB].at[0],
                                 semi[(u + 3) % NIB])
                pltpu.async_copy(dstf.at[pl.ds((w * NCH + j + 3) * C, C)],
                                 idxd[(u + 3) % NIB].at[0],
                                 semi[(u + 3) % NIB])

            up = (u + 5) % 6
            bp = (u + 2) % 3

            @pl.when(j >= 1)
            def _():
                pltpu.make_async_copy(xs.at[idxs[up].at[0]], rows[bp],
                                      semg[bp]).wait()
                pltpu.async_copy(rows[bp], acc.at[idxd[up].at[0]], sems[bp],
                                 add=True)

    pltpu.make_async_copy(xs.at[idxs[(NCH - 1) % 6].at[0]],
                          rows[(NCH - 1) % 3], semg[(NCH - 1) % 3]).wait()
    pltpu.async_copy(rows[(NCH - 1) % 3],
                     acc.at[idxd[(NCH - 1) % 6].at[0]],
                     sems[(NCH - 1) % 3], add=True)
    for b in range(3):
        pltpu.make_async_copy(rows[b], acc.at[idxd[b].at[0]], sems[b]).wait()

    plsc.subcore_barrier()
    pltpu.sync_copy(acc.at[pl.ds(s * RPTA, RPTA)],
                    part.at[c, pl.ds(s * RPTA, RPTA)])


@functools.lru_cache(maxsize=None)
def _agg_kernel():
    mesh = plsc.VectorSubcoreMesh(core_axis_name="c", subcore_axis_name="s",
                                  num_cores=NC, num_subcores=NS)
    return pl.kernel(
        _agg_body,
        out_type=jax.ShapeDtypeStruct((NC, NP, F), jnp.float32),
        mesh=mesh,
        scratch_types=(
            [pltpu.VMEM_SHARED((NACC, F), jnp.float32)]
            + [pltpu.VMEM((1, C), jnp.int32) for _ in range(2 * NIB)]
            + [pltpu.VMEM((C, F), jnp.float32) for _ in range(NRB)]
            + [pltpu.SemaphoreType.DMA for _ in range(NIB + 2 * NRB)]
        ),
    )


_BM = 1024
_GRID = -(-N // _BM)
_DB = _BM // 128


def _dcol(dv):
    g = lax.broadcasted_iota(jnp.int32, (_BM, _DB), 0) // 128
    h = lax.broadcasted_iota(jnp.int32, (_BM, _DB), 1)
    rep = jnp.dot((g == h).astype(jnp.float32), dv,
                  preferred_element_type=jnp.float32)
    ri = lax.broadcasted_iota(jnp.int32, (_BM, 128), 0)
    li = lax.broadcasted_iota(jnp.int32, (_BM, 128), 1)
    pick = li == (ri % 128)
    return jnp.sum(jnp.where(pick, rep, 0.0), axis=1, keepdims=True)


def _scale_body(degp, x, dinv, xs):
    d = degp[0] + degp[1] + 1.0
    dv = lax.rsqrt(d)
    dinv[...] = dv
    xs[...] = x[...] * _dcol(dv)


def _scale_call(degp, x):
    return pl.pallas_call(
        _scale_body,
        grid=(_GRID,),
        in_specs=[
            pl.BlockSpec((NC, _DB, 128), lambda i: (0, i, 0)),
            pl.BlockSpec((_BM, F), lambda i: (i, 0)),
        ],
        out_specs=[
            pl.BlockSpec((_DB, 128), lambda i: (i, 0)),
            pl.BlockSpec((_BM, F), lambda i: (i, 0)),
        ],
        out_shape=[
            jax.ShapeDtypeStruct((NP // 128, 128), jnp.float32),
            jax.ShapeDtypeStruct((N, F), jnp.float32),
        ],
    )(degp, x)


def _mid_body(part, xs, dinv, W1, b1, W2, ts):
    dv = _dcol(dinv[...])
    agg = (part[0] + part[1] + xs[...]) * dv
    h1 = jnp.maximum(
        jnp.dot(agg, W1[...], preferred_element_type=jnp.float32) + b1[...],
        0.0)
    t = jnp.dot(h1, W2[...], preferred_element_type=jnp.float32)
    ts[...] = t * dv


def _mid_call(part, xs, dinv, W1, b1, W2):
    return pl.pallas_call(
        _mid_body,
        grid=(_GRID,),
        in_specs=[
            pl.BlockSpec((NC, _BM, F), lambda i: (0, i, 0)),
            pl.BlockSpec((_BM, F), lambda i: (i, 0)),
            pl.BlockSpec((_DB, 128), lambda i: (i, 0)),
            pl.BlockSpec((F, 2 * F), lambda i: (0, 0)),
            pl.BlockSpec((1, 2 * F), lambda i: (0, 0)),
            pl.BlockSpec((2 * F, F), lambda i: (0, 0)),
        ],
        out_specs=pl.BlockSpec((_BM, F), lambda i: (i, 0)),
        out_shape=jax.ShapeDtypeStruct((N, F), jnp.float32),
    )(part, xs, dinv, W1, b1, W2)


def _fin_body(part, ts, dinv, b2, Wl, bl, out):
    dv = _dcol(dinv[...])
    h2 = jnp.maximum((part[0] + part[1] + ts[...]) * dv + b2[...], 0.0)
    lg = jnp.dot(h2, Wl[...], preferred_element_type=jnp.float32) + bl[...]
    m = jnp.max(lg, axis=-1, keepdims=True)
    lse = m + jnp.log(jnp.sum(jnp.exp(lg - m), axis=-1, keepdims=True))
    out[...] = lg - lse


def _fin_call(part, ts, dinv, b2, Wl, bl):
    return pl.pallas_call(
        _fin_body,
        grid=(_GRID,),
        in_specs=[
            pl.BlockSpec((NC, _BM, F), lambda i: (0, i, 0)),
            pl.BlockSpec((_BM, F), lambda i: (i, 0)),
            pl.BlockSpec((_DB, 128), lambda i: (i, 0)),
            pl.BlockSpec((1, F), lambda i: (0, 0)),
            pl.BlockSpec((F, 2), lambda i: (0, 0)),
            pl.BlockSpec((1, 2), lambda i: (0, 0)),
        ],
        out_specs=pl.BlockSpec((_BM, 2), lambda i: (i, 0)),
        out_shape=jax.ShapeDtypeStruct((N, 2), jnp.float32),
    )(part, ts, dinv, b2, Wl, bl)


def kernel(x, edge_index, W1, b1, W2, b2, Wl, bl):
    src = edge_index[0]
    dst = edge_index[1]
    pad = E_PAD - E
    ar = jnp.arange(pad, dtype=jnp.int32)
    srcf = jnp.concatenate([src, ar % 128])
    dstf = jnp.concatenate([dst, N + (ar % 32)])

    degp = _deg_kernel()(dstf)
    dinv, xs = _scale_call(degp.reshape(NC, NP // 128, 128), x)

    part1 = _agg_kernel()(xs, srcf, dstf)
    ts = _mid_call(part1, xs, dinv, W1, b1.reshape(1, 2 * F), W2)

    part2 = _agg_kernel()(ts, srcf, dstf)
    return _fin_call(part2, ts, dinv, b2.reshape(1, F), Wl, bl.reshape(1, 2))
